# baseline (speedup 1.0000x reference)
import functools

import jax
import jax.numpy as jnp
from jax import lax
from jax.experimental import pallas as pl
from jax.experimental.pallas import tpu as pltpu
from jax.experimental.pallas import tpu_sc as plsc

VOCAB = 100000
HID = 768
B = 4
S = 8192
EPS = 1e-12

N = B * S
NC = 2
NS = 16
NW = NC * NS
TPW = N // NW
CH = 16
NCH = TPW // CH
L = 16
NV = HID // L


def _tec_body(ids_hbm, tt_hbm, word_hbm, pos_hbm, type_hbm, gamma_hbm,
              beta_hbm, out_hbm, idx_v, ttv, wrows, prows, orows,
              type_v, g_v, b_v, sem):
    cid = lax.axis_index("c")
    sid = lax.axis_index("s")
    wid = sid * NC + cid
    base = wid * TPW

    pltpu.sync_copy(type_hbm, type_v)
    pltpu.sync_copy(gamma_hbm, g_v)
    pltpu.sync_copy(beta_hbm, b_v)

    def chunk(c, carry):
        tok0 = base + c * CH
        pos0 = lax.rem(tok0, S)
        pltpu.sync_copy(ids_hbm.at[pl.ds(tok0, CH)], idx_v)
        pltpu.sync_copy(tt_hbm.at[pl.ds(tok0, CH)], ttv)
        gather = pltpu.async_copy(word_hbm.at[idx_v], wrows, sem)
        pltpu.sync_copy(pos_hbm.at[pl.ds(pos0, CH)], prows)
        gather.wait()
        fvec = ttv[...].astype(jnp.float32)
        lane = lax.iota(jnp.int32, L)

        def token(t, tcarry):
            f = jnp.sum(jnp.where(lane == t, fvec, 0.0))
            acc_s = jnp.zeros((L,), jnp.float32)
            acc_q = jnp.zeros((L,), jnp.float32)
            for j in range(NV):
                sl = pl.ds(j * L, L)
                trow = type_v[0, sl] + f * (type_v[1, sl] - type_v[0, sl])
                x = wrows[t, sl] + prows[t, sl] + trow
                acc_s = acc_s + x
                acc_q = acc_q + x * x
                orows[t, sl] = x
            s1 = jnp.sum(acc_s)
            s2 = jnp.sum(acc_q)
            mean = s1 * (1.0 / HID)
            var = s2 * (1.0 / HID) - mean * mean
            v = var + EPS
            bits = lax.bitcast_convert_type(v, jnp.int32)
            y = lax.bitcast_convert_type(
                jnp.int32(0x5F3759DF) - (bits >> 1), jnp.float32)
            for _ in range(3):
                y = y * (1.5 - 0.5 * v * y * y)
            mean_v = jnp.full((L,), mean, jnp.float32)
            rstd_v = jnp.full((L,), y, jnp.float32)
            for j in range(NV):
                sl = pl.ds(j * L, L)
                orows[t, sl] = ((orows[t, sl] - mean_v) * rstd_v * g_v[sl]
                                + b_v[sl])
            return tcarry

        lax.fori_loop(0, CH, token, 0)
        pltpu.sync_copy(orows, out_hbm.at[pl.ds(tok0, CH)])
        return carry

    lax.fori_loop(0, NCH, chunk, 0)


@jax.jit
def _bert_embed_sc(ids_flat, tt_flat, word_table, pos_table, type_table,
                   ln_gamma, ln_beta):
    mesh = plsc.VectorSubcoreMesh(core_axis_name="c", subcore_axis_name="s")
    kern = functools.partial(
        pl.kernel,
        mesh=mesh,
        compiler_params=pltpu.CompilerParams(needs_layout_passes=False),
        out_type=jax.ShapeDtypeStruct((N, HID), jnp.float32),
        scratch_types=[
            pltpu.VMEM((CH,), jnp.int32),
            pltpu.VMEM((CH,), jnp.int32),
            pltpu.VMEM((CH, HID), jnp.float32),
            pltpu.VMEM((CH, HID), jnp.float32),
            pltpu.VMEM((CH, HID), jnp.float32),
            pltpu.VMEM((2, HID), jnp.float32),
            pltpu.VMEM((HID,), jnp.float32),
            pltpu.VMEM((HID,), jnp.float32),
            pltpu.SemaphoreType.DMA,
        ],
    )(_tec_body)
    return kern(ids_flat, tt_flat, word_table, pos_table, type_table,
                ln_gamma, ln_beta)


def kernel(input_ids, token_type_ids, word_table, pos_table, type_table,
           ln_gamma, ln_beta):
    ids_flat = input_ids.reshape(-1).astype(jnp.int32)
    tt_flat = token_type_ids.reshape(-1).astype(jnp.int32)
    out = _bert_embed_sc(ids_flat, tt_flat, word_table, pos_table,
                         type_table, ln_gamma, ln_beta)
    return out.reshape(B, S, HID)


# speedup vs baseline: 1.7700x; 1.7700x over previous
import functools

import jax
import jax.numpy as jnp
from jax import lax
from jax.experimental import pallas as pl
from jax.experimental.pallas import tpu as pltpu
from jax.experimental.pallas import tpu_sc as plsc

VOCAB = 100000
HID = 768
B = 4
S = 8192
EPS = 1e-12

N = B * S
NC = 2
NS = 16
NW = NC * NS
TPW = N // NW
CH = 16
NCH = TPW // CH
L = 16
NV = HID // L
TG = 8


def _tec_body(ids_hbm, tt_hbm, word_hbm, pos_hbm, type_hbm, gamma_hbm,
              beta_hbm, out_hbm, idx_v, ttv, wbuf, pbuf, obuf, type_v,
              g_v, b_v, sg0, sg1, sp0, sp1, so0, so1, si0, si1):
    cid = lax.axis_index("c")
    sid = lax.axis_index("s")
    wid = sid * NC + cid
    base = wid * TPW
    sg = (sg0, sg1)
    sp = (sp0, sp1)
    so = (so0, so1)
    si = (si0, si1)

    pltpu.sync_copy(type_hbm, type_v)
    pltpu.sync_copy(gamma_hbm, g_v)
    pltpu.sync_copy(beta_hbm, b_v)

    def tok0_of(c):
        return jnp.minimum(base + c * CH, N - CH)

    def issue_ids(c, b):
        t0 = tok0_of(c)
        pltpu.async_copy(ids_hbm.at[pl.ds(t0, CH)], idx_v.at[b], si[b])
        pltpu.async_copy(tt_hbm.at[pl.ds(t0, CH)], ttv.at[b], si[b])

    def wait_ids(b):
        pltpu.make_async_copy(ids_hbm.at[pl.ds(0, CH)], idx_v.at[b],
                              si[b]).wait()
        pltpu.make_async_copy(tt_hbm.at[pl.ds(0, CH)], ttv.at[b],
                              si[b]).wait()

    def issue_pos(c, b):
        p0 = lax.rem(tok0_of(c), S)
        pltpu.async_copy(pos_hbm.at[pl.ds(p0, CH)], pbuf.at[b], sp[b])

    def wait_pos(b):
        pltpu.make_async_copy(pos_hbm.at[pl.ds(0, CH)], pbuf.at[b],
                              sp[b]).wait()

    def issue_gather(b):
        pltpu.async_copy(word_hbm.at[idx_v.at[b]], wbuf.at[b], sg[b])

    def wait_gather(b):
        pltpu.make_async_copy(word_hbm.at[idx_v.at[b]], wbuf.at[b],
                              sg[b]).wait()

    def issue_out(c, b):
        pltpu.async_copy(obuf.at[b], out_hbm.at[pl.ds(base + c * CH, CH)],
                         so[b])

    def wait_out(b):
        pltpu.make_async_copy(obuf.at[b], out_hbm.at[pl.ds(0, CH)],
                              so[b]).wait()

    lane = lax.iota(jnp.int32, L)

    def compute(b):
        ttvec = ttv[b, :]
        tt_nz = ttvec != 0
        for t0g in range(0, CH, TG):
            fvs = []
            for t in range(TG):
                m = (lane == (t0g + t)) & tt_nz
                fvs.append(
                    plsc.all_reduce_population_count(m).astype(jnp.float32))
            accs = [jnp.zeros((L,), jnp.float32) for _ in range(TG)]
            accq = [jnp.zeros((L,), jnp.float32) for _ in range(TG)]
            for j in range(NV):
                sl = pl.ds(j * L, L)
                ty0 = type_v[0, sl]
                tyd = type_v[1, sl] - ty0
                for t in range(TG):
                    r = t0g + t
                    x = (wbuf[b, r, sl] + pbuf[b, r, sl]
                         + (ty0 + fvs[t] * tyd))
                    accs[t] = accs[t] + x
                    accq[t] = accq[t] + x * x
                    obuf[b, r, sl] = x
            means = []
            rstds = []
            for t in range(TG):
                s1 = jnp.sum(accs[t])
                s2 = jnp.sum(accq[t])
                mean = s1 * (1.0 / HID)
                var = s2 * (1.0 / HID) - mean * mean + EPS
                bits = lax.bitcast_convert_type(var, jnp.int32)
                y = lax.bitcast_convert_type(
                    jnp.int32(0x5F3759DF) - (bits >> 1), jnp.float32)
                for _ in range(3):
                    y = y * (1.5 - 0.5 * var * y * y)
                means.append(jnp.full((L,), mean, jnp.float32))
                rstds.append(jnp.full((L,), y, jnp.float32))
            for j in range(NV):
                sl = pl.ds(j * L, L)
                gv = g_v[sl]
                bv = b_v[sl]
                for t in range(TG):
                    r = t0g + t
                    obuf[b, r, sl] = ((obuf[b, r, sl] - means[t])
                                      * rstds[t] * gv + bv)

    issue_ids(0, 0)
    issue_ids(1, 1)
    wait_ids(0)
    issue_gather(0)
    issue_pos(0, 0)

    @pl.loop(0, NCH, step=2)
    def pair(c0):
        for bb in range(2):
            c = c0 + bb
            cur, nxt = bb, 1 - bb
            wait_gather(cur)
            wait_pos(cur)
            wait_ids(nxt)
            issue_gather(nxt)
            issue_pos(c + 1, nxt)
            @pl.when(c0 >= 2)
            def _wait_prev_out():
                wait_out(cur)
            compute(cur)
            issue_out(c, cur)
            issue_ids(c + 2, cur)

    wait_gather(0)
    wait_pos(0)
    wait_ids(1)
    wait_out(0)
    wait_out(1)


@jax.jit
def _bert_embed_sc(ids_flat, tt_flat, word_table, pos_table, type_table,
                   ln_gamma, ln_beta):
    mesh = plsc.VectorSubcoreMesh(core_axis_name="c", subcore_axis_name="s")
    kern = functools.partial(
        pl.kernel,
        mesh=mesh,
        compiler_params=pltpu.CompilerParams(needs_layout_passes=False),
        out_type=jax.ShapeDtypeStruct((N, HID), jnp.float32),
        scratch_types=[
            pltpu.VMEM((2, CH), jnp.int32),
            pltpu.VMEM((2, CH), jnp.int32),
            pltpu.VMEM((2, CH, HID), jnp.float32),
            pltpu.VMEM((2, CH, HID), jnp.float32),
            pltpu.VMEM((2, CH, HID), jnp.float32),
            pltpu.VMEM((2, HID), jnp.float32),
            pltpu.VMEM((HID,), jnp.float32),
            pltpu.VMEM((HID,), jnp.float32),
            pltpu.SemaphoreType.DMA,
            pltpu.SemaphoreType.DMA,
            pltpu.SemaphoreType.DMA,
            pltpu.SemaphoreType.DMA,
            pltpu.SemaphoreType.DMA,
            pltpu.SemaphoreType.DMA,
            pltpu.SemaphoreType.DMA,
            pltpu.SemaphoreType.DMA,
        ],
    )(_tec_body)
    return kern(ids_flat, tt_flat, word_table, pos_table, type_table,
                ln_gamma, ln_beta)


def kernel(input_ids, token_type_ids, word_table, pos_table, type_table,
           ln_gamma, ln_beta):
    ids_flat = input_ids.reshape(-1).astype(jnp.int32)
    tt_flat = token_type_ids.reshape(-1).astype(jnp.int32)
    out = _bert_embed_sc(ids_flat, tt_flat, word_table, pos_table,
                         type_table, ln_gamma, ln_beta)
    return out.reshape(B, S, HID)
